# Initial kernel scaffold; baseline (speedup 1.0000x reference)
import jax
import jax.numpy as jnp
from jax.experimental import pallas as pl


def kernel(x, edge_index, edge_attr, params):
    raise NotImplementedError("write your pallas kernel here")


# speedup vs baseline: 3.0749x; 3.0749x over previous
import jax
import jax.numpy as jnp
from jax import lax
from jax.experimental import pallas as pl
from jax.experimental.pallas import tpu as pltpu
from jax.experimental.pallas import tpu_sc as plsc

N = 10000
E = 320000
D = 128

NC = 2
NS = 16
NW = NC * NS
EPW = E // NW
C = 80
NCH = EPW // C
NP = 10240
RPT = NP // NS

BN = 2000
BE = 2000

f32 = jnp.float32


def _silu(z):
    return z * lax.logistic(z)


def _layernorm(r, g, b):
    m = jnp.mean(r, axis=-1, keepdims=True)
    v = jnp.mean((r - m) * (r - m), axis=-1, keepdims=True)
    return (r - m) * lax.rsqrt(v + 1e-5) * g + b


def _row_spec(blk, width):
    return pl.BlockSpec((blk, width), lambda i: (i, 0))


def _full_spec(shape):
    return pl.BlockSpec(shape, lambda i: (0,) * len(shape))


def _enc_body(x_ref, w1, b1, w2, b2, ws, wd, h_ref, a_ref, b_ref):
    pre = jnp.dot(x_ref[...], w1[...], preferred_element_type=f32) + b1[...]
    h = jnp.dot(_silu(pre), w2[...], preferred_element_type=f32) + b2[...]
    h_ref[...] = h
    a_ref[...] = jnp.dot(h, ws[...], preferred_element_type=f32)
    b_ref[...] = jnp.dot(h, wd[...], preferred_element_type=f32)


def _edge0_body(gs_ref, gd_ref, at_ref, we1, be1, we2, be2,
                w1e, b1, w2, b2, g, bln, out_ref):
    epre = jnp.dot(at_ref[...], we1[...], preferred_element_type=f32) + be1[...]
    e = jnp.dot(_silu(epre), we2[...], preferred_element_type=f32) + be2[...]
    pre = gs_ref[...] + gd_ref[...] + jnp.dot(
        e, w1e[...], preferred_element_type=f32) + b1[...]
    r = jnp.dot(_silu(pre), w2[...], preferred_element_type=f32) + b2[...] + e
    out_ref[...] = _layernorm(r, g[...], bln[...])


def _edge_body(gs_ref, gd_ref, e_ref, w1e, b1, w2, b2, g, bln, out_ref):
    e = e_ref[...]
    pre = gs_ref[...] + gd_ref[...] + jnp.dot(
        e, w1e[...], preferred_element_type=f32) + b1[...]
    r = jnp.dot(_silu(pre), w2[...], preferred_element_type=f32) + b2[...] + e
    out_ref[...] = _layernorm(r, g[...], bln[...])


def _node_body(h_ref, p0_ref, p1_ref, wh, wa, b1, w2, b2, g, bln, ws, wd,
               h_out, a_out, b_out):
    h = h_ref[...]
    agg = p0_ref[...] + p1_ref[...]
    pre = (jnp.dot(h, wh[...], preferred_element_type=f32)
           + jnp.dot(agg, wa[...], preferred_element_type=f32) + b1[...])
    r = jnp.dot(_silu(pre), w2[...], preferred_element_type=f32) + b2[...] + h
    hn = _layernorm(r, g[...], bln[...])
    h_out[...] = hn
    a_out[...] = jnp.dot(hn, ws[...], preferred_element_type=f32)
    b_out[...] = jnp.dot(hn, wd[...], preferred_element_type=f32)


def _node_last_body(h_ref, p0_ref, p1_ref, wh, wa, b1, w2, b2, g, bln, h_out):
    h = h_ref[...]
    agg = p0_ref[...] + p1_ref[...]
    pre = (jnp.dot(h, wh[...], preferred_element_type=f32)
           + jnp.dot(agg, wa[...], preferred_element_type=f32) + b1[...])
    r = jnp.dot(_silu(pre), w2[...], preferred_element_type=f32) + b2[...] + h
    h_out[...] = _layernorm(r, g[...], bln[...])


def _dec_body(h_ref, w1, b1, w2, b2, out_ref):
    pre = jnp.dot(h_ref[...], w1[...], preferred_element_type=f32) + b1[...]
    out_ref[...] = jnp.dot(_silu(pre), w2[...], preferred_element_type=f32) + b2[...]


def _gather_body(a_hbm, b_hbm, src_hbm, dst_hbm, gs_hbm, gd_hbm,
                 idx_s, idx_d, rows_s, rows_d, sem1, sem2, sem3, sem4):
    cid = lax.axis_index("c")
    sid = lax.axis_index("s")
    base = (sid * NC + cid) * EPW

    def step(i, carry):
        off = base + i * C
        ci1 = pltpu.async_copy(src_hbm.at[pl.ds(off, C)], idx_s, sem1)
        ci2 = pltpu.async_copy(dst_hbm.at[pl.ds(off, C)], idx_d, sem2)
        ci1.wait()
        ci2.wait()
        cg1 = pltpu.async_copy(a_hbm.at[idx_s], rows_s, sem1)
        cg2 = pltpu.async_copy(b_hbm.at[idx_d], rows_d, sem2)
        cg1.wait()
        cg2.wait()
        co1 = pltpu.async_copy(rows_s, gs_hbm.at[pl.ds(off, C)], sem3)
        co2 = pltpu.async_copy(rows_d, gd_hbm.at[pl.ds(off, C)], sem4)
        co1.wait()
        co2.wait()
        return carry

    lax.fori_loop(0, NCH, step, 0)


def _scatter_body(ne_hbm, dst_hbm, zero_hbm, p_hbm,
                  idx_d, rows, acc, sem1, sem2):
    cid = lax.axis_index("c")
    sid = lax.axis_index("s")
    base = (sid * NC + cid) * EPW
    r0 = sid * RPT

    for j in range(RPT // C):
        pltpu.sync_copy(zero_hbm.at[pl.ds(r0 + j * C, C)], rows)
        pltpu.sync_copy(rows, acc.at[pl.ds(r0 + j * C, C)])
    plsc.subcore_barrier()

    def step(i, carry):
        off = base + i * C
        c1 = pltpu.async_copy(dst_hbm.at[pl.ds(off, C)], idx_d, sem1)
        c2 = pltpu.async_copy(ne_hbm.at[pl.ds(off, C)], rows, sem2)
        c1.wait()
        c2.wait()
        pltpu.sync_copy(rows, acc.at[idx_d], add=True)
        return carry

    lax.fori_loop(0, NCH, step, 0)
    plsc.subcore_barrier()

    for j in range(RPT // C):
        pltpu.sync_copy(acc.at[pl.ds(r0 + j * C, C)], rows)
        pltpu.sync_copy(rows, p_hbm.at[cid, pl.ds(r0 + j * C, C)])


_sc_mesh = plsc.VectorSubcoreMesh(core_axis_name="c", subcore_axis_name="s")

_gather = pl.kernel(
    _gather_body,
    out_type=[jax.ShapeDtypeStruct((E, D), f32),
              jax.ShapeDtypeStruct((E, D), f32)],
    mesh=_sc_mesh,
    scratch_types=[
        pltpu.VMEM((C,), jnp.int32), pltpu.VMEM((C,), jnp.int32),
        pltpu.VMEM((C, D), f32), pltpu.VMEM((C, D), f32),
        pltpu.SemaphoreType.DMA, pltpu.SemaphoreType.DMA,
        pltpu.SemaphoreType.DMA, pltpu.SemaphoreType.DMA,
    ],
)

_scatter = pl.kernel(
    _scatter_body,
    out_type=jax.ShapeDtypeStruct((NC, NP, D), f32),
    mesh=_sc_mesh,
    scratch_types=[
        pltpu.VMEM((C,), jnp.int32), pltpu.VMEM((C, D), f32),
        pltpu.VMEM_SHARED((NP, D), f32),
        pltpu.SemaphoreType.DMA, pltpu.SemaphoreType.DMA,
    ],
)


def _mm_call(body, n_rows, blk, in_widths, out_widths, weight_shapes):
    grid = (n_rows // blk,)
    in_specs = ([_row_spec(blk, w) for w in in_widths]
                + [_full_spec(s) for s in weight_shapes])
    outs = [jax.ShapeDtypeStruct((n_rows, w), f32) for w in out_widths]
    out_specs = [_row_spec(blk, w) for w in out_widths]
    if len(outs) == 1:
        outs, out_specs = outs[0], out_specs[0]
    return pl.pallas_call(body, grid=grid, in_specs=in_specs,
                          out_specs=out_specs, out_shape=outs)


def kernel(x, edge_index, edge_attr, params):
    src = edge_index[0]
    dst = edge_index[1]
    attr8 = jnp.pad(edge_attr, ((0, 0), (0, 4)))

    lyrs = params["layers"]
    w1s = [l["edge_mlp"][0]["W"][0:D] for l in lyrs]
    w1d = [l["edge_mlp"][0]["W"][D:2 * D] for l in lyrs]
    w1e = [l["edge_mlp"][0]["W"][2 * D:3 * D] for l in lyrs]
    eb1 = [l["edge_mlp"][0]["b"].reshape(1, D) for l in lyrs]
    ew2 = [l["edge_mlp"][1]["W"] for l in lyrs]
    eb2 = [l["edge_mlp"][1]["b"].reshape(1, D) for l in lyrs]
    eg = [l["edge_norm"]["g"].reshape(1, D) for l in lyrs]
    ebn = [l["edge_norm"]["b"].reshape(1, D) for l in lyrs]
    nwh = [l["node_mlp"][0]["W"][0:D] for l in lyrs]
    nwa = [l["node_mlp"][0]["W"][D:2 * D] for l in lyrs]
    nb1 = [l["node_mlp"][0]["b"].reshape(1, D) for l in lyrs]
    nw2 = [l["node_mlp"][1]["W"] for l in lyrs]
    nb2 = [l["node_mlp"][1]["b"].reshape(1, D) for l in lyrs]
    ng = [l["node_norm"]["g"].reshape(1, D) for l in lyrs]
    nbn = [l["node_norm"]["b"].reshape(1, D) for l in lyrs]

    ne = params["node_enc"]
    ee = params["edge_enc"]
    we1 = jnp.pad(ee[0]["W"], ((0, 4), (0, 0)))
    be1 = ee[0]["b"].reshape(1, D)
    we2 = ee[1]["W"]
    be2 = ee[1]["b"].reshape(1, D)

    fd, hd, sd = params["flow_dec"], params["heat_dec"], params["species_dec"]
    wd1 = jnp.pad(jnp.concatenate(
        [fd[0]["W"], hd[0]["W"], sd[0]["W"]], axis=1), ((0, 0), (0, 2)))
    bd1 = jnp.pad(jnp.concatenate(
        [fd[0]["b"], hd[0]["b"], sd[0]["b"]]), (0, 2)).reshape(1, 8)
    wd2 = jnp.zeros((8, 8), f32)
    wd2 = wd2.at[0:4, 0:4].set(fd[1]["W"])
    wd2 = wd2.at[4:5, 4:5].set(hd[1]["W"])
    wd2 = wd2.at[5:6, 5:6].set(sd[1]["W"])
    bd2 = jnp.pad(jnp.concatenate(
        [fd[1]["b"], hd[1]["b"], sd[1]["b"]]), (0, 2)).reshape(1, 8)

    zeros_nd = jnp.zeros((NP, D), f32)

    h, a, b = _mm_call(_enc_body, N, BN, [D], [D, D, D],
                       [(D, D), (1, D), (D, D), (1, D), (D, D), (D, D)])(
        x, ne[0]["W"], ne[0]["b"].reshape(1, D), ne[1]["W"],
        ne[1]["b"].reshape(1, D), w1s[0], w1d[0])

    e = None
    for i in range(4):
        gs, gd = _gather(a, b, src, dst)
        if i == 0:
            e = _mm_call(_edge0_body, E, BE, [D, D, 8], [D],
                         [(8, D), (1, D), (D, D), (1, D),
                          (D, D), (1, D), (D, D), (1, D), (1, D), (1, D)])(
                gs, gd, attr8, we1, be1, we2, be2,
                w1e[0], eb1[0], ew2[0], eb2[0], eg[0], ebn[0])
        else:
            e = _mm_call(_edge_body, E, BE, [D, D, D], [D],
                         [(D, D), (1, D), (D, D), (1, D), (1, D), (1, D)])(
                gs, gd, e, w1e[i], eb1[i], ew2[i], eb2[i], eg[i], ebn[i])
        p = _scatter(e, dst, zeros_nd)
        p0, p1 = p[0, :N], p[1, :N]
        if i < 3:
            h, a, b = _mm_call(_node_body, N, BN, [D, D, D], [D, D, D],
                               [(D, D), (D, D), (1, D), (D, D), (1, D),
                                (1, D), (1, D), (D, D), (D, D)])(
                h, p0, p1, nwh[i], nwa[i], nb1[i], nw2[i], nb2[i],
                ng[i], nbn[i], w1s[i + 1], w1d[i + 1])
        else:
            h = _mm_call(_node_last_body, N, BN, [D, D, D], [D],
                         [(D, D), (D, D), (1, D), (D, D), (1, D),
                          (1, D), (1, D)])(
                h, p0, p1, nwh[i], nwa[i], nb1[i], nw2[i], nb2[i],
                ng[i], nbn[i])

    out = _mm_call(_dec_body, N, BN, [D], [8],
                   [(D, 8), (1, 8), (8, 8), (1, 8)])(h, wd1, bd1, wd2, bd2)
    return out[:, 0:4], out[:, 4:5], out[:, 5:6]
